# baseline (speedup 1.0000x reference)
import jax
import jax.numpy as jnp
from jax.experimental import pallas as pl
from jax.experimental.pallas import tpu as pltpu

_ALPHA = 0.5
_BB = 8


def _align_body(x_ref, pt_ref, o_ref, p2_ref):
    @pl.when(pl.program_id(0) == 0)
    def _():
        ptv = pt_ref[...]
        p2_ref[...] = jnp.sum(ptv * ptv, axis=0, keepdims=True)

    xb = x_ref[...]
    hw = xb.shape[2]
    feat = jnp.sum(xb, axis=2) * (1.0 / hw)
    f2 = jnp.sum(feat * feat, axis=1, keepdims=True)
    dots = jax.lax.dot_general(
        feat, pt_ref[...], (((1,), (0,)), ((), ())),
        preferred_element_type=jnp.float32)
    d2 = jnp.maximum((f2 + p2_ref[...]) - 2.0 * dots, 0.0)
    m = jnp.min(d2, axis=1, keepdims=True)
    ii = jax.lax.broadcasted_iota(jnp.int32, d2.shape, 1)
    idx = jnp.min(jnp.where(d2 <= m, ii, jnp.int32(d2.shape[1])),
                  axis=1, keepdims=True)
    onehot = (ii == idx).astype(jnp.float32)
    nearest_t = jax.lax.dot_general(
        pt_ref[...], onehot, (((1,), (1,)), ((), ())),
        preferred_element_type=jnp.float32)
    nearest = nearest_t.T
    delta = _ALPHA * (nearest - feat)
    o_ref[...] = xb + delta[:, :, None]


def kernel(x, prototypes):
    B, C, H, W = x.shape
    K = prototypes.shape[0]
    HW = H * W
    x3 = x.reshape(B, C, HW)
    pt = prototypes.T
    out3 = pl.pallas_call(
        _align_body,
        grid=(B // _BB,),
        in_specs=[
            pl.BlockSpec((_BB, C, HW), lambda i: (i, 0, 0)),
            pl.BlockSpec((C, K), lambda i: (0, 0)),
        ],
        out_specs=pl.BlockSpec((_BB, C, HW), lambda i: (i, 0, 0)),
        out_shape=jax.ShapeDtypeStruct((B, C, HW), x.dtype),
        scratch_shapes=[pltpu.VMEM((1, K), jnp.float32)],
        compiler_params=pltpu.CompilerParams(
            dimension_semantics=("arbitrary",)),
    )(x3, pt)
    return out3.reshape(B, C, H, W)


# speedup vs baseline: 1.0691x; 1.0691x over previous
import jax
import jax.numpy as jnp
from jax.experimental import pallas as pl
from jax.experimental.pallas import tpu as pltpu

_ALPHA = 0.5
_BB = 8


def _align_body(x_ref, pt_ref, p_ref, p2_ref, o_ref):
    xb = x_ref[...]
    hw = xb.shape[2]
    feat = jnp.sum(xb, axis=2) * (1.0 / hw)
    f2 = jnp.sum(feat * feat, axis=1, keepdims=True)
    dots = jax.lax.dot_general(
        feat.astype(jnp.bfloat16), pt_ref[...], (((1,), (0,)), ((), ())),
        preferred_element_type=jnp.float32)
    d2 = jnp.maximum((f2 + p2_ref[...]) - 2.0 * dots, 0.0)
    m = jnp.min(d2, axis=1, keepdims=True)
    ii = jax.lax.broadcasted_iota(jnp.int32, d2.shape, 1)
    idx = jnp.min(jnp.where(d2 <= m, ii, jnp.int32(d2.shape[1])),
                  axis=1, keepdims=True)
    onehot = (ii == idx).astype(jnp.bfloat16)
    nearest = jax.lax.dot_general(
        onehot, p_ref[...], (((1,), (0,)), ((), ())),
        preferred_element_type=jnp.float32)
    delta = _ALPHA * (nearest - feat)
    o_ref[...] = xb + delta[:, :, None]


def kernel(x, prototypes):
    B, C, H, W = x.shape
    K = prototypes.shape[0]
    HW = H * W
    x3 = x.reshape(B, C, HW)
    pt_bf = prototypes.T.astype(jnp.bfloat16)
    p_bf = prototypes.astype(jnp.bfloat16)
    p2 = jnp.sum(prototypes * prototypes, axis=1)[None, :]
    out3 = pl.pallas_call(
        _align_body,
        grid=(B // _BB,),
        in_specs=[
            pl.BlockSpec((_BB, C, HW), lambda i: (i, 0, 0)),
            pl.BlockSpec((C, K), lambda i: (0, 0)),
            pl.BlockSpec((K, C), lambda i: (0, 0)),
            pl.BlockSpec((1, K), lambda i: (0, 0)),
        ],
        out_specs=pl.BlockSpec((_BB, C, HW), lambda i: (i, 0, 0)),
        out_shape=jax.ShapeDtypeStruct((B, C, HW), x.dtype),
        compiler_params=pltpu.CompilerParams(
            dimension_semantics=("parallel",)),
    )(x3, pt_bf, p_bf, p2)
    return out3.reshape(B, C, H, W)


# speedup vs baseline: 2.4288x; 2.2719x over previous
import jax
import jax.numpy as jnp
from jax.experimental import pallas as pl
from jax.experimental.pallas import tpu as pltpu

_ALPHA = 0.5
_BB = 8


def _align_body(x_ref, p_ref, p2_ref, o_ref):
    xb = x_ref[...]
    hw = xb.shape[1]
    feat = jnp.sum(xb, axis=1) * (1.0 / hw)
    f2 = jnp.sum(feat * feat, axis=1, keepdims=True)
    dots = jax.lax.dot_general(
        feat.astype(jnp.bfloat16), p_ref[...], (((1,), (1,)), ((), ())),
        preferred_element_type=jnp.float32)
    d2 = jnp.maximum((f2 + p2_ref[...]) - 2.0 * dots, 0.0)
    m = jnp.min(d2, axis=1, keepdims=True)
    ii = jax.lax.broadcasted_iota(jnp.int32, d2.shape, 1)
    idx = jnp.min(jnp.where(d2 <= m, ii, jnp.int32(d2.shape[1])),
                  axis=1, keepdims=True)
    onehot = (ii == idx).astype(jnp.bfloat16)
    nearest = jax.lax.dot_general(
        onehot, p_ref[...], (((1,), (0,)), ((), ())),
        preferred_element_type=jnp.float32)
    delta = _ALPHA * (nearest - feat)
    o_ref[...] = xb + delta[:, None, :]


def kernel(x, prototypes):
    B, C, H, W = x.shape
    K = prototypes.shape[0]
    HW = H * W
    xt = x.transpose(0, 2, 3, 1).reshape(B, HW, C)
    p_bf = prototypes.astype(jnp.bfloat16)
    p2 = jnp.sum(prototypes * prototypes, axis=1)[None, :]
    out_t = pl.pallas_call(
        _align_body,
        grid=(B // _BB,),
        in_specs=[
            pl.BlockSpec((_BB, HW, C), lambda i: (i, 0, 0)),
            pl.BlockSpec((K, C), lambda i: (0, 0)),
            pl.BlockSpec((1, K), lambda i: (0, 0)),
        ],
        out_specs=pl.BlockSpec((_BB, HW, C), lambda i: (i, 0, 0)),
        out_shape=jax.ShapeDtypeStruct((B, HW, C), x.dtype),
        compiler_params=pltpu.CompilerParams(
            dimension_semantics=("parallel",)),
    )(xt, p_bf, p2)
    return out_t.reshape(B, H, W, C).transpose(0, 3, 1, 2)
